# baseline (speedup 1.0000x reference)
import functools

import jax
import jax.numpy as jnp
from jax import lax
from jax.experimental import pallas as pl
from jax.experimental.pallas import tpu as pltpu
from jax.experimental.pallas import tpu_sc as plsc

B = 32
C = 4
N = 24
L = 8
LAT = 48
LON = 96
NE = LAT * LON
HID = 128
OUT_LEN = 24
NC = 2
NS = 16
NW = NC * NS
KV = NE // 16


def _wid():
    return lax.axis_index("s") * NC + lax.axis_index("c")


def _sc_argmin_body(clat_hbm, clon_hbm, csta_hbm,
                    j_out, exlat_out, exlon_out,
                    clat_v, clon_v, csta_v, jv, latv, lonv):
    wid = _wid()

    @pl.when(wid < N)
    def _():
        pltpu.sync_copy(clat_hbm, clat_v)
        pltpu.sync_copy(clon_hbm, clon_v)
        pltpu.sync_copy(csta_hbm, csta_v)
        slat = plsc.load_gather(
            csta_v, [jnp.full((16,), 2 * wid, jnp.int32)])
        slon = plsc.load_gather(
            csta_v, [jnp.full((16,), 2 * wid + 1, jnp.int32)])

        def body(k, carry):
            best, bidx = carry
            cl = clat_v[pl.ds(k * 16, 16)]
            cn = clon_v[pl.ds(k * 16, 16)]
            dl = cl - slat
            dn = cn - slon
            d = dl * dl + dn * dn
            idx = lax.iota(jnp.int32, 16) + k * 16
            upd = d < best
            return jnp.where(upd, d, best), jnp.where(upd, idx, bidx)

        best, bidx = lax.fori_loop(
            0, KV, body,
            (jnp.full((16,), 1e30, jnp.float32),
             jnp.zeros((16,), jnp.int32)))
        m = jnp.min(best)
        jm = jnp.min(jnp.where(best == m, bidx, jnp.int32(1 << 30)))
        jsplat = jnp.full((16,), jm, jnp.int32)
        jv[...] = jsplat
        latv[...] = plsc.load_gather(clat_v, [jsplat])
        lonv[...] = plsc.load_gather(clon_v, [jsplat])
        pltpu.sync_copy(jv, j_out.at[pl.ds(wid * 16, 16)])
        pltpu.sync_copy(latv, exlat_out.at[pl.ds(wid * 16, 16)])
        pltpu.sync_copy(lonv, exlon_out.at[pl.ds(wid * 16, 16)])


def _make_sc_argmin():
    return functools.partial(
        pl.kernel,
        out_type=(jax.ShapeDtypeStruct((NW * 16,), jnp.int32),
                  jax.ShapeDtypeStruct((NW * 16,), jnp.float32),
                  jax.ShapeDtypeStruct((NW * 16,), jnp.float32)),
        mesh=plsc.VectorSubcoreMesh(
            core_axis_name="c", subcore_axis_name="s",
            num_cores=NC, num_subcores=NS),
        compiler_params=pltpu.CompilerParams(needs_layout_passes=False),
        scratch_types=[
            pltpu.VMEM((NE,), jnp.float32),
            pltpu.VMEM((NE,), jnp.float32),
            pltpu.VMEM((2 * N,), jnp.float32),
            pltpu.VMEM((16,), jnp.int32),
            pltpu.VMEM((16,), jnp.float32),
            pltpu.VMEM((16,), jnp.float32),
        ])(_sc_argmin_body)


def _sc_gather_body(j_hbm, era_hbm, pan_hbm,
                    era_out, pan_out,
                    j_v, eidx, pidx, erows, prows, sem1, sem2):
    wid = _wid()
    pltpu.sync_copy(j_hbm, j_v)
    for i6 in range(6):
        iv = lax.iota(jnp.int32, 16) + i6 * 16
        c = iv // N
        n = iv % N
        j = plsc.load_gather(j_v, [n * 16])
        lat = j // LON
        lon = j - lat * LON
        erow = (wid * C + c) * (LAT * (LON + 1)) + lat * (LON + 1) + lon
        prow = (wid * C + c) * NE + j
        eidx[pl.ds(i6 * 16, 16)] = erow
        pidx[pl.ds(i6 * 16, 16)] = prow
    cp1 = pltpu.async_copy(era_hbm.at[eidx], erows, sem1)
    cp2 = pltpu.async_copy(pan_hbm.at[pidx], prows, sem2)
    cp1.wait()
    cp2.wait()
    pltpu.sync_copy(erows, era_out.at[pl.ds(wid * C * N, C * N)])
    pltpu.sync_copy(prows, pan_out.at[pl.ds(wid * C * N, C * N)])


def _make_sc_gather():
    return functools.partial(
        pl.kernel,
        out_type=(jax.ShapeDtypeStruct((B * C * N, L), jnp.float32),
                  jax.ShapeDtypeStruct((B * C * N, L), jnp.float32)),
        mesh=plsc.VectorSubcoreMesh(
            core_axis_name="c", subcore_axis_name="s",
            num_cores=NC, num_subcores=NS),
        compiler_params=pltpu.CompilerParams(
            needs_layout_passes=False, use_tc_tiling_on_sc=False),
        scratch_types=[
            pltpu.VMEM((NW * 16,), jnp.int32),
            pltpu.VMEM((C * N,), jnp.int32),
            pltpu.VMEM((C * N,), jnp.int32),
            pltpu.VMEM((C * N, L), jnp.float32),
            pltpu.VMEM((C * N, L), jnp.float32),
            pltpu.SemaphoreType.DMA,
            pltpu.SemaphoreType.DMA,
        ])(_sc_gather_body)


def _mlp_body(*refs):
    it = iter(refs)
    nxt = lambda: next(it)[...]
    mm = lambda a, b: jnp.dot(a, b, preferred_element_type=jnp.float32)

    def tile24(p):
        return jnp.broadcast_to(p[None], (B, N, HID)).reshape(B * N, HID)

    obs = [nxt() for _ in range(C)]
    era = [nxt() for _ in range(C)]
    pan = [nxt() for _ in range(C)]
    clat = nxt(); clon = nxt(); exlat = nxt(); exlon = nxt()
    dlat = exlat - clat
    dlon = exlon - clon

    embWc = [nxt() for _ in range(C)]
    eW32 = nxt(); eW33 = nxt(); eb1 = nxt(); eW2 = nxt(); eb2 = nxt()
    acc = mm(obs[0], embWc[0])
    for cc in range(1, C):
        acc = acc + mm(obs[cc], embWc[cc])
    p_emb = clon * eW32 + clat * eW33
    x = jnp.tanh(acc + tile24(p_emb) + eb1)
    x = jnp.tanh(mm(x, eW2) + eb2)

    for _ in range(2):
        m1h = nxt()
        wera = [nxt() for _ in range(C)]
        wpan = [nxt() for _ in range(C)]
        wlon = nxt(); wlat = nxt(); mb1 = nxt()
        mW2 = nxt(); mb2 = nxt()
        u1a = nxt(); u1b = nxt(); ub1 = nxt(); uW2 = nxt(); ub2 = nxt()
        acc = mm(x, m1h)
        for cc in range(C):
            acc = acc + mm(era[cc], wera[cc]) + mm(pan[cc], wpan[cc])
        p_pos = dlon * wlon + dlat * wlat
        h = jnp.tanh(acc + tile24(p_pos) + mb1)
        h = jnp.tanh(mm(h, mW2) + mb2)
        o = jnp.tanh(mm(x, u1a) + mm(h, u1b) + ub1)
        x = mm(o, uW2) + ub2

    oW1 = nxt(); ob1 = nxt(); oW2 = nxt(); ob2 = nxt()
    out_ref = next(it)
    out_ref[...] = mm(jnp.tanh(mm(x, oW1) + ob1), oW2) + ob2


def kernel(obs_his, era_his, pan_fut, csta, cera, cpan,
           emb_W1, emb_b1, emb_W2, emb_b2,
           ex1_mW1, ex1_mb1, ex1_mW2, ex1_mb2,
           ex1_uW1, ex1_ub1, ex1_uW2, ex1_ub2,
           ex2_mW1, ex2_mb1, ex2_mW2, ex2_mb2,
           ex2_uW1, ex2_ub1, ex2_uW2, ex2_ub2,
           out_W1, out_b1, out_W2, out_b2):
    cand = cera[:, :-1, :].reshape(NE, 2)
    cand_lat = cand[:, 0]
    cand_lon = cand[:, 1]
    csta_flat = csta.reshape(2 * N)
    era_tab = era_his.reshape(B * C * LAT * (LON + 1), L)
    pan_tab = pan_fut.reshape(B * C * NE, L)

    j_out, exlat_o, exlon_o = _make_sc_argmin()(cand_lat, cand_lon, csta_flat)
    era_g, pan_g = _make_sc_gather()(j_out, era_tab, pan_tab)

    exlat = exlat_o.reshape(NW, 16)[:N, 0:1]
    exlon = exlon_o.reshape(NW, 16)[:N, 0:1]
    clat = csta[:, 0:1]
    clon = csta[:, 1:2]

    def cmats(x4):
        return [x4[:, cc].reshape(B * N, L) for cc in range(C)]

    obs_c = cmats(obs_his)
    era_c = cmats(era_g.reshape(B, C, N, L))
    pan_c = cmats(pan_g.reshape(B, C, N, L))

    def wsplit(W, base):
        return [W[base:base + C * L][cc::C] for cc in range(C)]

    row = lambda W, r: W[r:r + 1]
    vec = lambda b: b.reshape(1, -1)

    args = []
    args += obs_c + era_c + pan_c
    args += [clat, clon, exlat, exlon]
    args += wsplit(emb_W1, 0)
    args += [row(emb_W1, C * L), row(emb_W1, C * L + 1),
             vec(emb_b1), emb_W2, vec(emb_b2)]
    for mW1, mb1, mW2, mb2, uW1, ub1, uW2, ub2 in (
            (ex1_mW1, ex1_mb1, ex1_mW2, ex1_mb2,
             ex1_uW1, ex1_ub1, ex1_uW2, ex1_ub2),
            (ex2_mW1, ex2_mb1, ex2_mW2, ex2_mb2,
             ex2_uW1, ex2_ub1, ex2_uW2, ex2_ub2)):
        args += [mW1[:HID]]
        args += wsplit(mW1, HID)
        args += wsplit(mW1, HID + C * L)
        args += [row(mW1, HID + 2 * C * L), row(mW1, HID + 2 * C * L + 1),
                 vec(mb1), mW2, vec(mb2),
                 uW1[:HID], uW1[HID:], vec(ub1), uW2, vec(ub2)]
    args += [out_W1, vec(out_b1), out_W2, vec(out_b2)]

    out = pl.pallas_call(
        _mlp_body,
        out_shape=jax.ShapeDtypeStruct((B * N, OUT_LEN), jnp.float32),
    )(*args)
    return out.reshape(B, N, OUT_LEN)[:, None]


# speedup vs baseline: 2.7243x; 2.7243x over previous
import functools

import jax
import jax.numpy as jnp
from jax import lax
from jax.experimental import pallas as pl
from jax.experimental.pallas import tpu as pltpu
from jax.experimental.pallas import tpu_sc as plsc

B = 32
C = 4
N = 24
L = 8
LAT = 48
LON = 96
NE = LAT * LON
HID = 128
OUT_LEN = 24
NC = 2
NS = 16
NW = NC * NS
KV = NE // 16


def _sc_argmin_body(clat_hbm, clon_hbm, csta_hbm,
                    j_out, exlat_out, exlon_out,
                    clat_v, clon_v, csta_v, jv, latv, lonv):
    wid = lax.axis_index("s") * NC + lax.axis_index("c")

    @pl.when(wid < N)
    def _():
        pltpu.sync_copy(clat_hbm, clat_v)
        pltpu.sync_copy(clon_hbm, clon_v)
        pltpu.sync_copy(csta_hbm, csta_v)
        slat = plsc.load_gather(
            csta_v, [jnp.full((16,), 2 * wid, jnp.int32)])
        slon = plsc.load_gather(
            csta_v, [jnp.full((16,), 2 * wid + 1, jnp.int32)])

        def body(k, carry):
            best, bidx = carry
            cl = clat_v[pl.ds(k * 16, 16)]
            cn = clon_v[pl.ds(k * 16, 16)]
            dl = cl - slat
            dn = cn - slon
            d = dl * dl + dn * dn
            idx = lax.iota(jnp.int32, 16) + k * 16
            upd = d < best
            return jnp.where(upd, d, best), jnp.where(upd, idx, bidx)

        best, bidx = lax.fori_loop(
            0, KV, body,
            (jnp.full((16,), 1e30, jnp.float32),
             jnp.zeros((16,), jnp.int32)))
        m = jnp.min(best)
        jm = jnp.min(jnp.where(best == m, bidx, jnp.int32(1 << 30)))
        jsplat = jnp.full((16,), jm, jnp.int32)
        jv[...] = jsplat
        latv[...] = plsc.load_gather(clat_v, [jsplat])
        lonv[...] = plsc.load_gather(clon_v, [jsplat])
        pltpu.sync_copy(jv, j_out.at[pl.ds(wid * 16, 16)])
        pltpu.sync_copy(latv, exlat_out.at[pl.ds(wid * 16, 16)])
        pltpu.sync_copy(lonv, exlon_out.at[pl.ds(wid * 16, 16)])


def _make_sc_argmin():
    return functools.partial(
        pl.kernel,
        out_type=(jax.ShapeDtypeStruct((NW * 16,), jnp.int32),
                  jax.ShapeDtypeStruct((NW * 16,), jnp.float32),
                  jax.ShapeDtypeStruct((NW * 16,), jnp.float32)),
        mesh=plsc.VectorSubcoreMesh(
            core_axis_name="c", subcore_axis_name="s",
            num_cores=NC, num_subcores=NS),
        compiler_params=pltpu.CompilerParams(needs_layout_passes=False),
        scratch_types=[
            pltpu.VMEM((NE,), jnp.float32),
            pltpu.VMEM((NE,), jnp.float32),
            pltpu.VMEM((2 * N,), jnp.float32),
            pltpu.VMEM((16,), jnp.int32),
            pltpu.VMEM((16,), jnp.float32),
            pltpu.VMEM((16,), jnp.float32),
        ])(_sc_argmin_body)


def _mlp_body(*refs):
    it = iter(refs)
    era_hbm = next(it)
    pan_hbm = next(it)
    j_smem = next(it)
    nref = lambda: next(it)
    nxt = lambda: next(it)[...]
    mm = lambda a, b: jnp.dot(a, b, preferred_element_type=jnp.float32)

    def tile24(p):
        return jnp.broadcast_to(p[None], (B, N, HID)).reshape(B * N, HID)

    obs = [nxt() for _ in range(C)]
    clat = nxt(); clon = nxt(); exlat = nxt(); exlon = nxt()

    embWc = [nxt() for _ in range(C)]
    eW32 = nxt(); eW33 = nxt(); eb1 = nxt(); eW2 = nxt(); eb2 = nxt()
    layers = []
    for _ in range(2):
        layers.append(dict(
            m1h=nref(), wera=[nref() for _ in range(C)],
            wpan=[nref() for _ in range(C)],
            wlon=nref(), wlat=nref(), mb1=nref(),
            mW2=nref(), mb2=nref(), u1a=nref(), u1b=nref(),
            ub1=nref(), uW2=nref(), ub2=nref()))
    oW1 = nref(); ob1 = nref(); oW2 = nref(); ob2 = nref()
    out_ref = next(it)
    era_sc = next(it)
    pan_sc = next(it)
    sem = next(it)

    cps = []
    for n in range(N):
        j = j_smem[n * 16]
        lat = j // LON
        lon = j - lat * LON
        cps.append(pltpu.async_copy(
            era_hbm.at[:, :, lat, lon, :], era_sc.at[:, :, n, :], sem))
        cps.append(pltpu.async_copy(
            pan_hbm.at[:, :, lat, lon, :], pan_sc.at[:, :, n, :], sem))

    dlat = exlat[...] - clat[...]
    dlon = exlon[...] - clon[...]
    acc = mm(obs[0], embWc[0])
    for cc in range(1, C):
        acc = acc + mm(obs[cc], embWc[cc])
    p_emb = clon[...] * eW32 + clat[...] * eW33
    x = jnp.tanh(acc + tile24(p_emb) + eb1)
    x = jnp.tanh(mm(x, eW2) + eb2)

    for cp in cps:
        cp.wait()
    era = [era_sc[:, cc].reshape(B * N, L) for cc in range(C)]
    pan = [pan_sc[:, cc].reshape(B * N, L) for cc in range(C)]

    for ly in layers:
        acc = mm(x, ly['m1h'][...])
        for cc in range(C):
            acc = acc + mm(era[cc], ly['wera'][cc][...])
            acc = acc + mm(pan[cc], ly['wpan'][cc][...])
        p_pos = dlon * ly['wlon'][...] + dlat * ly['wlat'][...]
        h = jnp.tanh(acc + tile24(p_pos) + ly['mb1'][...])
        h = jnp.tanh(mm(h, ly['mW2'][...]) + ly['mb2'][...])
        o = jnp.tanh(mm(x, ly['u1a'][...]) + mm(h, ly['u1b'][...])
                     + ly['ub1'][...])
        x = mm(o, ly['uW2'][...]) + ly['ub2'][...]

    out_ref[...] = mm(jnp.tanh(mm(x, oW1[...]) + ob1[...]),
                      oW2[...]) + ob2[...]


def kernel(obs_his, era_his, pan_fut, csta, cera, cpan,
           emb_W1, emb_b1, emb_W2, emb_b2,
           ex1_mW1, ex1_mb1, ex1_mW2, ex1_mb2,
           ex1_uW1, ex1_ub1, ex1_uW2, ex1_ub2,
           ex2_mW1, ex2_mb1, ex2_mW2, ex2_mb2,
           ex2_uW1, ex2_ub1, ex2_uW2, ex2_ub2,
           out_W1, out_b1, out_W2, out_b2):
    cand = cera[:, :-1, :].reshape(NE, 2)
    cand_lat = cand[:, 0]
    cand_lon = cand[:, 1]
    csta_flat = csta.reshape(2 * N)

    j_out, exlat_o, exlon_o = _make_sc_argmin()(cand_lat, cand_lon, csta_flat)

    exlat = exlat_o.reshape(NW, 16)[:N, 0:1]
    exlon = exlon_o.reshape(NW, 16)[:N, 0:1]
    clat = csta[:, 0:1]
    clon = csta[:, 1:2]
    obs_c = [obs_his[:, cc].reshape(B * N, L) for cc in range(C)]

    def wsplit(W, base):
        return [W[base:base + C * L][cc::C] for cc in range(C)]

    row = lambda W, r: W[r:r + 1]
    vec = lambda b: b.reshape(1, -1)

    args = [era_his, pan_fut, j_out]
    args += obs_c
    args += [clat, clon, exlat, exlon]
    args += wsplit(emb_W1, 0)
    args += [row(emb_W1, C * L), row(emb_W1, C * L + 1),
             vec(emb_b1), emb_W2, vec(emb_b2)]
    for mW1, mb1, mW2, mb2, uW1, ub1, uW2, ub2 in (
            (ex1_mW1, ex1_mb1, ex1_mW2, ex1_mb2,
             ex1_uW1, ex1_ub1, ex1_uW2, ex1_ub2),
            (ex2_mW1, ex2_mb1, ex2_mW2, ex2_mb2,
             ex2_uW1, ex2_ub1, ex2_uW2, ex2_ub2)):
        args += [mW1[:HID]]
        args += wsplit(mW1, HID)
        args += wsplit(mW1, HID + C * L)
        args += [row(mW1, HID + 2 * C * L), row(mW1, HID + 2 * C * L + 1),
                 vec(mb1), mW2, vec(mb2),
                 uW1[:HID], uW1[HID:], vec(ub1), uW2, vec(ub2)]
    args += [out_W1, vec(out_b1), out_W2, vec(out_b2)]

    n_in = len(args)
    in_specs = [pl.BlockSpec(memory_space=pltpu.HBM),
                pl.BlockSpec(memory_space=pltpu.HBM),
                pl.BlockSpec(memory_space=pltpu.SMEM)]
    in_specs += [pl.BlockSpec(memory_space=pltpu.VMEM)
                 for _ in range(n_in - 3)]

    out = pl.pallas_call(
        _mlp_body,
        out_shape=jax.ShapeDtypeStruct((B * N, OUT_LEN), jnp.float32),
        in_specs=in_specs,
        out_specs=pl.BlockSpec(memory_space=pltpu.VMEM),
        scratch_shapes=[
            pltpu.VMEM((B, C, N, L), jnp.float32),
            pltpu.VMEM((B, C, N, L), jnp.float32),
            pltpu.SemaphoreType.DMA,
        ],
    )(*args)
    return out.reshape(B, N, OUT_LEN)[:, None]


# speedup vs baseline: 21.2707x; 7.8076x over previous
import functools

import jax
import jax.numpy as jnp
from jax import lax
from jax.experimental import pallas as pl
from jax.experimental.pallas import tpu as pltpu
from jax.experimental.pallas import tpu_sc as plsc

B = 32
C = 4
N = 24
L = 8
LAT = 48
LON = 96
NE = LAT * LON
HID = 128
OUT_LEN = 24
NC = 2
NS = 16
NW = NC * NS
KV = NE // 16


def _sc_argmin_body(clat_hbm, clon_hbm, csta_hbm,
                    j_out, dlat_out, dlon_out,
                    clat_v, clon_v, csta_v, jv, latv, lonv):
    wid = lax.axis_index("s") * NC + lax.axis_index("c")

    @pl.when(wid < N)
    def _():
        pltpu.sync_copy(clat_hbm, clat_v)
        pltpu.sync_copy(clon_hbm, clon_v)
        pltpu.sync_copy(csta_hbm, csta_v)
        slat = plsc.load_gather(
            csta_v, [jnp.full((16,), 2 * wid, jnp.int32)])
        slon = plsc.load_gather(
            csta_v, [jnp.full((16,), 2 * wid + 1, jnp.int32)])

        def body(k, carry):
            best, bidx = carry
            for u in range(4):
                off = k * 64 + u * 16
                cl = clat_v[pl.ds(off, 16)]
                cn = clon_v[pl.ds(off, 16)]
                dl = cl - slat
                dn = cn - slon
                d = dl * dl + dn * dn
                idx = lax.iota(jnp.int32, 16) + off
                upd = d < best
                best = jnp.where(upd, d, best)
                bidx = jnp.where(upd, idx, bidx)
            return best, bidx

        best, bidx = lax.fori_loop(
            0, KV // 4, body,
            (jnp.full((16,), 1e30, jnp.float32),
             jnp.zeros((16,), jnp.int32)))
        m = jnp.min(best)
        jm = jnp.min(jnp.where(best == m, bidx, jnp.int32(1 << 30)))
        jsplat = jnp.full((16,), jm, jnp.int32)
        jv[...] = jsplat
        latv[...] = plsc.load_gather(clat_v, [jsplat]) - slat
        lonv[...] = plsc.load_gather(clon_v, [jsplat]) - slon
        pltpu.sync_copy(jv, j_out.at[pl.ds(wid * 16, 16)])
        pltpu.sync_copy(latv, dlat_out.at[pl.ds(wid * 16, 16)])
        pltpu.sync_copy(lonv, dlon_out.at[pl.ds(wid * 16, 16)])


def _make_sc_argmin():
    return functools.partial(
        pl.kernel,
        out_type=(jax.ShapeDtypeStruct((NW * 16,), jnp.int32),
                  jax.ShapeDtypeStruct((NW * 16,), jnp.float32),
                  jax.ShapeDtypeStruct((NW * 16,), jnp.float32)),
        mesh=plsc.VectorSubcoreMesh(
            core_axis_name="c", subcore_axis_name="s",
            num_cores=NC, num_subcores=NS),
        compiler_params=pltpu.CompilerParams(needs_layout_passes=False),
        scratch_types=[
            pltpu.VMEM((NE,), jnp.float32),
            pltpu.VMEM((NE,), jnp.float32),
            pltpu.VMEM((2 * N,), jnp.float32),
            pltpu.VMEM((16,), jnp.int32),
            pltpu.VMEM((16,), jnp.float32),
            pltpu.VMEM((16,), jnp.float32),
        ])(_sc_argmin_body)


def _mlp_body(era_hbm, pan_hbm, j_smem, dlat_smem, dlon_smem, csta_smem,
              obs_r,
              eW1_r, eb1_r, eW2_r, eb2_r,
              m1W1_r, m1b1_r, m1W2_r, m1b2_r, u1W1_r, u1b1_r, u1W2_r, u1b2_r,
              m2W1_r, m2b1_r, m2W2_r, m2b2_r, u2W1_r, u2b1_r, u2W2_r, u2b2_r,
              oW1_r, ob1_r, oW2t_r, ob2_r,
              out_ref, era_sc, pan_sc, sem):
    mm = lambda a, b: jnp.dot(a, b, preferred_element_type=jnp.float32)

    cps = []
    lons = []
    for n in range(N):
        j = j_smem[n * 16]
        lat = j // LON
        lons.append(j - lat * LON)
        cps.append(pltpu.async_copy(
            era_hbm.at[:, :, lat], era_sc.at[:, :, n], sem))
        cps.append(pltpu.async_copy(
            pan_hbm.at[:, :, lat], pan_sc.at[:, :, n], sem))

    def col24(vals, dtype):
        return jnp.concatenate(
            [jnp.full((1, 1), v, dtype) for v in vals], axis=0)

    def tile24(p):
        return jnp.broadcast_to(p[None], (B, N, HID)).reshape(B * N, HID)

    def row_of(W, r):
        oh = (lax.broadcasted_iota(jnp.int32, (1, W.shape[0]), 1)
              == r).astype(jnp.float32)
        return mm(oh, W)

    def psel(blk, cc):
        t = lax.broadcasted_iota(jnp.int32, (L, C * L), 0)
        r = lax.broadcasted_iota(jnp.int32, (L, C * L), 1)
        oh = (r == t * C + cc).astype(jnp.float32)
        return mm(oh, blk)

    clat = col24([csta_smem[n, 0] for n in range(N)], jnp.float32)
    clon = col24([csta_smem[n, 1] for n in range(N)], jnp.float32)
    dlat = col24([dlat_smem[n * 16] for n in range(N)], jnp.float32)
    dlon = col24([dlon_smem[n * 16] for n in range(N)], jnp.float32)

    obs4 = jnp.transpose(obs_r[...], (0, 1, 3, 2))
    obs = [obs4[:, cc].reshape(B * N, L) for cc in range(C)]

    eW1 = eW1_r[...]
    eblk = eW1[0:C * L]
    acc = mm(obs[0], psel(eblk, 0))
    for cc in range(1, C):
        acc = acc + mm(obs[cc], psel(eblk, cc))
    p_emb = clon * row_of(eW1, C * L) + clat * row_of(eW1, C * L + 1)
    x = jnp.tanh(acc + tile24(p_emb) + eb1_r[...].reshape(1, -1))
    x = jnp.tanh(mm(x, eW2_r[...]) + eb2_r[...].reshape(1, -1))

    for cp in cps:
        cp.wait()
    lonrow = jnp.concatenate(
        [jnp.full((1, 1), lo, jnp.int32) for lo in lons], axis=1)
    I24r = jnp.broadcast_to(
        (lax.broadcasted_iota(jnp.int32, (N, N), 0)
         == lax.broadcasted_iota(jnp.int32, (N, N), 1)
         ).astype(jnp.float32)[None, :, None, :],
        (B, N, 1, N)).reshape(B * N, 1, N)

    def pick(sc_ref, width, cc):
        ohT = (lax.broadcasted_iota(jnp.int32, (width, N), 0)
               == lonrow).astype(jnp.float32)
        g = mm(sc_ref[:, cc].reshape(B * N * L, width), ohT)
        return jnp.sum(g.reshape(B * N, L, N) * I24r, axis=-1)

    era = [pick(era_sc, LON + 1, cc) for cc in range(C)]
    pan = [pick(pan_sc, LON, cc) for cc in range(C)]

    for (mW1_r, mb1_r, mW2_r, mb2_r, uW1_r, ub1_r, uW2_r, ub2_r) in (
            (m1W1_r, m1b1_r, m1W2_r, m1b2_r, u1W1_r, u1b1_r, u1W2_r, u1b2_r),
            (m2W1_r, m2b1_r, m2W2_r, m2b2_r, u2W1_r, u2b1_r, u2W2_r, u2b2_r)):
        mW1 = mW1_r[...]
        eb = mW1[HID:HID + C * L]
        pb = mW1[HID + C * L:HID + 2 * C * L]
        acc = mm(x, mW1[0:HID])
        for cc in range(C):
            acc = acc + mm(era[cc], psel(eb, cc))
            acc = acc + mm(pan[cc], psel(pb, cc))
        p_pos = (dlon * row_of(mW1, HID + 2 * C * L)
                 + dlat * row_of(mW1, HID + 2 * C * L + 1))
        h = jnp.tanh(acc + tile24(p_pos) + mb1_r[...].reshape(1, -1))
        h = jnp.tanh(mm(h, mW2_r[...]) + mb2_r[...].reshape(1, -1))
        uW1 = uW1_r[...]
        o = jnp.tanh(mm(x, uW1[0:HID]) + mm(h, uW1[HID:2 * HID])
                     + ub1_r[...].reshape(1, -1))
        x = mm(o, uW2_r[...]) + ub2_r[...].reshape(1, -1)

    y = jnp.tanh(mm(x, oW1_r[...]) + ob1_r[...].reshape(1, -1))
    out = lax.dot_general(
        y, oW2t_r[...], (((1,), (1,)), ((), ())),
        preferred_element_type=jnp.float32) + ob2_r[...].reshape(1, -1)
    out_ref[...] = out.reshape(B, 1, N, OUT_LEN)


def kernel(obs_his, era_his, pan_fut, csta, cera, cpan,
           emb_W1, emb_b1, emb_W2, emb_b2,
           ex1_mW1, ex1_mb1, ex1_mW2, ex1_mb2,
           ex1_uW1, ex1_ub1, ex1_uW2, ex1_ub2,
           ex2_mW1, ex2_mb1, ex2_mW2, ex2_mb2,
           ex2_uW1, ex2_ub1, ex2_uW2, ex2_ub2,
           out_W1, out_b1, out_W2, out_b2):
    cand_lat = cera[:, :-1, 0].reshape(NE)
    cand_lon = cera[:, :-1, 1].reshape(NE)
    csta_flat = csta.reshape(2 * N)

    j_out, dlat_o, dlon_o = _make_sc_argmin()(cand_lat, cand_lon, csta_flat)

    era_t = jnp.swapaxes(era_his, 3, 4)
    pan_t = jnp.swapaxes(pan_fut, 3, 4)
    obs_t = jnp.swapaxes(obs_his, 2, 3)
    oW2t = out_W2.T

    hbm = pl.BlockSpec(memory_space=pltpu.HBM)
    smem = pl.BlockSpec(memory_space=pltpu.SMEM)
    vmem = pl.BlockSpec(memory_space=pltpu.VMEM)
    args = [era_t, pan_t, j_out, dlat_o, dlon_o, csta,
            obs_t,
            emb_W1, emb_b1, emb_W2, emb_b2,
            ex1_mW1, ex1_mb1, ex1_mW2, ex1_mb2,
            ex1_uW1, ex1_ub1, ex1_uW2, ex1_ub2,
            ex2_mW1, ex2_mb1, ex2_mW2, ex2_mb2,
            ex2_uW1, ex2_ub1, ex2_uW2, ex2_ub2,
            out_W1, out_b1, oW2t, out_b2]
    in_specs = [hbm, hbm, smem, smem, smem, smem] + [vmem] * (len(args) - 6)

    return pl.pallas_call(
        _mlp_body,
        out_shape=jax.ShapeDtypeStruct((B, 1, N, OUT_LEN), jnp.float32),
        in_specs=in_specs,
        out_specs=vmem,
        compiler_params=pltpu.CompilerParams(
            vmem_limit_bytes=56 * 1024 * 1024),
        scratch_shapes=[
            pltpu.VMEM((B, C, N, L, LON + 1), jnp.float32),
            pltpu.VMEM((B, C, N, L, LON), jnp.float32),
            pltpu.SemaphoreType.DMA,
        ],
    )(*args)
